# Initial kernel scaffold; baseline (speedup 1.0000x reference)
import jax
import jax.numpy as jnp
from jax.experimental import pallas as pl


def kernel(inputs, prior, layers):
    raise NotImplementedError("write your pallas kernel here")


# speedup vs baseline: 2.2660x; 2.2660x over previous
import jax
import jax.numpy as jnp
from jax.experimental import pallas as pl
from jax.experimental.pallas import tpu as pltpu

FEAT = 256
HALF = FEAT // 2
HIDDEN = FEAT * 2
K = 1024
BATCH = 8192
LAYERS = 4
BLOCK = 256


def _fused_body(x_ref, prior_t_ref, *rest):
    wrefs = rest[:4 * LAYERS]
    x_out_ref, loss_ref = rest[4 * LAYERS], rest[4 * LAYERS + 1]

    x = x_ref[...]
    xa = x[:, :HALF]
    xb = x[:, HALF:]
    jac = jnp.zeros((x.shape[0],), jnp.float32)
    for i in range(LAYERS):
        w1 = wrefs[4 * i][...]
        b1 = wrefs[4 * i + 1][...]
        w2 = wrefs[4 * i + 2][...]
        b2 = wrefs[4 * i + 3][...]
        h = jnp.maximum(
            jnp.dot(xa, w1, preferred_element_type=jnp.float32) + b1, 0.0)
        o = jnp.dot(h, w2, preferred_element_type=jnp.float32) + b2
        log_s = o[:, :HALF]
        t = o[:, HALF:]
        if i < LAYERS - 1:
            log_s = jnp.tanh(log_s)
        yb = xb * jnp.exp(log_s) + t
        jac = jac + jnp.sum(log_s, axis=1)
        xa, xb = yb, xa

    xfull = jnp.concatenate([xa, xb], axis=1)
    x_out_ref[...] = xfull

    prior_t = prior_t_ref[...]
    nb = jnp.sum(prior_t * prior_t, axis=0)
    scores = jnp.dot(xfull, prior_t,
                     preferred_element_type=jnp.float32)
    m = jnp.min(nb[None, :] - 2.0 * scores, axis=1)
    na = jnp.sum(xfull * xfull, axis=1)
    mind = jnp.maximum(na + m, 0.0)
    part = jnp.sum(0.625 * mind - jac)

    @pl.when(pl.program_id(0) == 0)
    def _init():
        loss_ref[0, 0] = 0.0

    loss_ref[0, 0] += part


def kernel(inputs, prior, layers):
    operands = [inputs, prior.T]
    w_specs = []
    for p in layers:
        operands += [p["W1"], p["b1"].reshape(1, HIDDEN),
                     p["W2"], p["b2"].reshape(1, FEAT)]
        w_specs += [
            pl.BlockSpec((HALF, HIDDEN), lambda i: (0, 0)),
            pl.BlockSpec((1, HIDDEN), lambda i: (0, 0)),
            pl.BlockSpec((HIDDEN, FEAT), lambda i: (0, 0)),
            pl.BlockSpec((1, FEAT), lambda i: (0, 0)),
        ]

    grid = (BATCH // BLOCK,)
    x_out, loss_sum = pl.pallas_call(
        _fused_body,
        grid=grid,
        in_specs=[
            pl.BlockSpec((BLOCK, FEAT), lambda i: (i, 0)),
            pl.BlockSpec((FEAT, K), lambda i: (0, 0)),
        ] + w_specs,
        out_specs=[
            pl.BlockSpec((BLOCK, FEAT), lambda i: (i, 0)),
            pl.BlockSpec((1, 1), lambda i: (0, 0),
                         memory_space=pltpu.SMEM),
        ],
        out_shape=[
            jax.ShapeDtypeStruct((BATCH, FEAT), jnp.float32),
            jax.ShapeDtypeStruct((1, 1), jnp.float32),
        ],
        compiler_params=pltpu.CompilerParams(
            dimension_semantics=("arbitrary",)),
    )(*operands)
    loss = (loss_sum[0, 0] / BATCH).astype(jnp.float32)
    return x_out, loss


# speedup vs baseline: 4.3119x; 1.9029x over previous
import jax
import jax.numpy as jnp
from jax.experimental import pallas as pl
from jax.experimental.pallas import tpu as pltpu

FEAT = 256
HALF = FEAT // 2
HIDDEN = FEAT * 2
K = 1024
BATCH = 8192
LAYERS = 4
BLOCK = 1024


def _fused_body(x_ref, prior_t_ref, *rest):
    wrefs = rest[:4 * LAYERS]
    x_out_ref, loss_ref = rest[4 * LAYERS], rest[4 * LAYERS + 1]

    x = x_ref[...]
    xa = x[:, :HALF]
    xb = x[:, HALF:]
    jac = jnp.zeros((x.shape[0],), jnp.float32)
    for i in range(LAYERS):
        w1 = wrefs[4 * i][...]
        b1 = wrefs[4 * i + 1][...]
        w2 = wrefs[4 * i + 2][...]
        b2 = wrefs[4 * i + 3][...]
        h = jnp.maximum(
            jnp.dot(xa, w1, preferred_element_type=jnp.float32) + b1, 0.0)
        o = jnp.dot(h, w2, preferred_element_type=jnp.float32) + b2
        log_s = o[:, :HALF]
        t = o[:, HALF:]
        if i < LAYERS - 1:
            log_s = jnp.tanh(log_s)
        yb = xb * jnp.exp(log_s) + t
        jac = jac + jnp.sum(log_s, axis=1)
        xa, xb = yb, xa

    xfull = jnp.concatenate([xa, xb], axis=1)
    x_out_ref[...] = xfull

    prior_t = prior_t_ref[...]
    nb = jnp.sum(prior_t * prior_t, axis=0)
    scores = jnp.dot(xfull, prior_t,
                     preferred_element_type=jnp.float32)
    m = jnp.min(nb[None, :] - 2.0 * scores, axis=1)
    na = jnp.sum(xfull * xfull, axis=1)
    mind = jnp.maximum(na + m, 0.0)
    part = jnp.sum(0.625 * mind - jac)

    @pl.when(pl.program_id(0) == 0)
    def _init():
        loss_ref[0, 0] = 0.0

    loss_ref[0, 0] += part


def kernel(inputs, prior, layers):
    operands = [inputs, prior.T]
    w_specs = []
    for p in layers:
        operands += [p["W1"], p["b1"].reshape(1, HIDDEN),
                     p["W2"], p["b2"].reshape(1, FEAT)]
        w_specs += [
            pl.BlockSpec((HALF, HIDDEN), lambda i: (0, 0)),
            pl.BlockSpec((1, HIDDEN), lambda i: (0, 0)),
            pl.BlockSpec((HIDDEN, FEAT), lambda i: (0, 0)),
            pl.BlockSpec((1, FEAT), lambda i: (0, 0)),
        ]

    grid = (BATCH // BLOCK,)
    x_out, loss_sum = pl.pallas_call(
        _fused_body,
        grid=grid,
        in_specs=[
            pl.BlockSpec((BLOCK, FEAT), lambda i: (i, 0)),
            pl.BlockSpec((FEAT, K), lambda i: (0, 0)),
        ] + w_specs,
        out_specs=[
            pl.BlockSpec((BLOCK, FEAT), lambda i: (i, 0)),
            pl.BlockSpec((1, 1), lambda i: (0, 0),
                         memory_space=pltpu.SMEM),
        ],
        out_shape=[
            jax.ShapeDtypeStruct((BATCH, FEAT), jnp.float32),
            jax.ShapeDtypeStruct((1, 1), jnp.float32),
        ],
        compiler_params=pltpu.CompilerParams(
            dimension_semantics=("arbitrary",)),
    )(*operands)
    loss = (loss_sum[0, 0] / BATCH).astype(jnp.float32)
    return x_out, loss
